# baseline (speedup 1.0000x reference)
import jax
import jax.numpy as jnp
from jax import lax
from jax.experimental import pallas as pl
from jax.experimental.pallas import tpu as pltpu
from jax.experimental.pallas import tpu_sc as plsc

TOPK = 8
_NC, _NS = 2, 16
_NW = _NC * _NS
_LANES = 16
_GRPS = 1


def _logits_block(x_ref, w_ref, b_ref, out_ref):
    x = x_ref[...]
    w = w_ref[...]
    logits = lax.dot_general(
        w.astype(jnp.bfloat16), x.astype(jnp.bfloat16),
        (((1,), (1,)), ((), ())),
        preferred_element_type=jnp.float32,
    )
    out_ref[0] = logits + b_ref[...]


def _topk_softmax_16(l_v, base, e_total):
    vals = [l_v[e, pl.ds(base, _LANES)] for e in range(e_total)]
    idxs = [jnp.full((_LANES,), e, jnp.int32) for e in range(e_total)]
    ms, ims = [], []
    for k in range(TOPK):
        tv, ti = vals, idxs
        while len(tv) > 1:
            nv, ni = [], []
            for a in range(0, len(tv), 2):
                ge = tv[a] >= tv[a + 1]
                nv.append(jnp.where(ge, tv[a], tv[a + 1]))
                ni.append(jnp.where(ge, ti[a], ti[a + 1]))
            tv, ti = nv, ni
        m, im = tv[0], ti[0]
        ms.append(m)
        ims.append(im)
        if k + 1 < TOPK:
            vals = [jnp.where(im == e, -jnp.inf, v)
                    for e, v in enumerate(vals)]
    ex = [jnp.full((_LANES,), 1.0, jnp.float32)]
    ex += [jnp.exp(mm - ms[0]) for mm in ms[1:]]
    denom = ex[0]
    for t in ex[1:]:
        denom = denom + t
    inv = 1.0 / denom
    return [e_ * inv for e_ in ex], ims


def _sc_topk(l_hbm, wts_hbm, idx_hbm, l_v, w_v, i_v, sem0, sem1):
    e_total = l_v.shape[0]
    bt = l_v.shape[1]
    half = bt // 2
    wid = lax.axis_index("s") * _NC + lax.axis_index("c")
    c0 = pltpu.async_copy(
        l_hbm.at[wid, :, pl.ds(0, half)], l_v.at[:, pl.ds(0, half)], sem0)
    c1 = pltpu.async_copy(
        l_hbm.at[wid, :, pl.ds(half, half)],
        l_v.at[:, pl.ds(half, half)], sem1)
    c0.wait()

    def run_range(lo, hi):
        @plsc.parallel_loop(lo, hi, 1, unroll=_GRPS)
        def body(j):
            base = j * _LANES
            wts, ims = _topk_softmax_16(l_v, base, e_total)
            for k in range(TOPK):
                w_v[k, pl.ds(base, _LANES)] = wts[k]
                i_v[k, pl.ds(base, _LANES)] = ims[k]

    run_range(0, half // _LANES)
    c1.wait()
    run_range(half // _LANES, bt // _LANES)
    pltpu.sync_copy(w_v, wts_hbm.at[wid])
    pltpu.sync_copy(i_v, idx_hbm.at[wid])


def kernel(x, W, b):
    B, S, H = x.shape
    E = W.shape[0]
    T = B * S
    BT = T // _NW
    xf = x.reshape(T, H)
    logits = pl.pallas_call(
        _logits_block,
        grid=(_NW,),
        in_specs=[
            pl.BlockSpec((BT, H), lambda i: (i, 0)),
            pl.BlockSpec((E, H), lambda i: (0, 0)),
            pl.BlockSpec((E, 1), lambda i: (0, 0)),
        ],
        out_specs=pl.BlockSpec((1, E, BT), lambda i: (i, 0, 0)),
        out_shape=jax.ShapeDtypeStruct((_NW, E, BT), jnp.float32),
        compiler_params=pltpu.CompilerParams(
            vmem_limit_bytes=60 * 1024 * 1024,
        ),
    )(xf, W, b.reshape(E, 1))

    sc_call = pl.kernel(
        _sc_topk,
        out_type=[
            jax.ShapeDtypeStruct((_NW, TOPK, BT), jnp.float32),
            jax.ShapeDtypeStruct((_NW, TOPK, BT), jnp.int32),
        ],
        mesh=plsc.VectorSubcoreMesh(
            core_axis_name="c", subcore_axis_name="s"),
        scratch_types=[
            pltpu.VMEM((E, BT), jnp.float32),
            pltpu.VMEM((TOPK, BT), jnp.float32),
            pltpu.VMEM((TOPK, BT), jnp.int32),
            pltpu.SemaphoreType.DMA,
            pltpu.SemaphoreType.DMA,
        ],
    )
    wts, idx = sc_call(logits)
    wts = wts.transpose(0, 2, 1).reshape(B, S, TOPK)
    idx = idx.transpose(0, 2, 1).reshape(B, S, TOPK)
    return (wts, idx)


# speedup vs baseline: 1.0219x; 1.0219x over previous
import jax
import jax.numpy as jnp
from jax import lax
from jax.experimental import pallas as pl
from jax.experimental.pallas import tpu as pltpu
from jax.experimental.pallas import tpu_sc as plsc

TOPK = 8
_NC, _NS = 2, 16
_NW = _NC * _NS
_LANES = 16
_GRPS = 1


def _logits_block(x_ref, w_ref, b_ref, out_ref):
    x = x_ref[...]
    w = w_ref[...]
    logits = lax.dot_general(
        w.astype(jnp.bfloat16), x.astype(jnp.bfloat16),
        (((1,), (1,)), ((), ())),
        preferred_element_type=jnp.float32,
    )
    out_ref[0] = logits + b_ref[...]


def _topk_softmax_16(l_v, base, e_total):
    vals = [l_v[e, pl.ds(base, _LANES)] for e in range(e_total)]
    idxs = [jnp.full((_LANES,), e, jnp.int32) for e in range(e_total)]
    ms, ims = [], []
    for k in range(TOPK):
        tv, ti = vals, idxs
        while len(tv) > 1:
            nv, ni = [], []
            for a in range(0, len(tv), 2):
                ge = tv[a] >= tv[a + 1]
                nv.append(jnp.where(ge, tv[a], tv[a + 1]))
                ni.append(jnp.where(ge, ti[a], ti[a + 1]))
            tv, ti = nv, ni
        m, im = tv[0], ti[0]
        ms.append(m)
        ims.append(im)
        if k + 1 < TOPK:
            vals = [jnp.where(im == e, -jnp.inf, v)
                    for e, v in enumerate(vals)]
    ex = [jnp.full((_LANES,), 1.0, jnp.float32)]
    ex += [jnp.exp(mm - ms[0]) for mm in ms[1:]]
    denom = ex[0]
    for t in ex[1:]:
        denom = denom + t
    inv = 1.0 / denom
    return [e_ * inv for e_ in ex], ims


def _sc_topk(l_hbm, wts_hbm, idx_hbm, l_v, w_v, i_v):
    e_total = l_v.shape[0]
    bt = l_v.shape[1]
    wid = lax.axis_index("s") * _NC + lax.axis_index("c")
    pltpu.sync_copy(l_hbm.at[wid], l_v)

    @plsc.parallel_loop(0, bt // _LANES, 1, unroll=_GRPS)
    def body(j):
        base = j * _LANES
        wts, ims = _topk_softmax_16(l_v, base, e_total)
        for k in range(TOPK):
            w_v[k, pl.ds(base, _LANES)] = wts[k]
            i_v[k, pl.ds(base, _LANES)] = ims[k]

    pltpu.sync_copy(w_v, wts_hbm.at[wid])
    pltpu.sync_copy(i_v, idx_hbm.at[wid])


def kernel(x, W, b):
    B, S, H = x.shape
    E = W.shape[0]
    T = B * S
    BT = T // _NW
    xf = x.reshape(T, H)
    logits = pl.pallas_call(
        _logits_block,
        grid=(_NW,),
        in_specs=[
            pl.BlockSpec((BT, H), lambda i: (i, 0)),
            pl.BlockSpec((E, H), lambda i: (0, 0)),
            pl.BlockSpec((E, 1), lambda i: (0, 0)),
        ],
        out_specs=pl.BlockSpec((1, E, BT), lambda i: (i, 0, 0)),
        out_shape=jax.ShapeDtypeStruct((_NW, E, BT), jnp.float32),
        compiler_params=pltpu.CompilerParams(
            vmem_limit_bytes=60 * 1024 * 1024,
        ),
    )(xf, W, b.reshape(E, 1))

    sc_call = pl.kernel(
        _sc_topk,
        out_type=[
            jax.ShapeDtypeStruct((_NW, TOPK, BT), jnp.float32),
            jax.ShapeDtypeStruct((_NW, TOPK, BT), jnp.int32),
        ],
        mesh=plsc.VectorSubcoreMesh(
            core_axis_name="c", subcore_axis_name="s"),
        scratch_types=[
            pltpu.VMEM((E, BT), jnp.float32),
            pltpu.VMEM((TOPK, BT), jnp.float32),
            pltpu.VMEM((TOPK, BT), jnp.int32),
        ],
    )
    wts, idx = sc_call(logits)
    wts = wts.transpose(0, 2, 1).reshape(B, S, TOPK)
    idx = idx.transpose(0, 2, 1).reshape(B, S, TOPK)
    return (wts, idx)
